# baseline (speedup 1.0000x reference)
import functools

import jax
import jax.numpy as jnp
from jax import lax
from jax.experimental import pallas as pl
from jax.experimental.pallas import tpu as pltpu
from jax.experimental.pallas import tpu_sc as plsc

N = 10000
E = 320000
D = 128
H = 8
C1 = 8
NCLS = 40

NC = 2
NS = 16
NW = NC * NS
EPW = E // NW
CH = 80
NCHUNK = EPW // CH
RPT = N // NS
ZR = 125

SRC_W = 80
DST_W = 32
SRC_W2 = 48
DST_W2 = 16

_NEG_SLOPE = 0.2


def _leaky(t):
    return jnp.where(t >= 0, t, _NEG_SLOPE * t)


def _tc1_body(x_ref, w1_ref, as_ref, ad_ref, ts_ref, td_ref):
    h = jnp.dot(x_ref[...], w1_ref[...], preferred_element_type=jnp.float32)
    a_s = jnp.dot(h, as_ref[...], preferred_element_type=jnp.float32)
    a_d = jnp.dot(h, ad_ref[...], preferred_element_type=jnp.float32)
    gmax = jnp.max(a_s, axis=0, keepdims=True)
    m = _leaky(a_d + gmax)
    ts_ref[...] = (jnp.pad(h, ((0, 0), (0, SRC_W - 64)))
                   + jnp.pad(a_s, ((0, 0), (64, SRC_W - 72))))
    td_ref[...] = (jnp.pad(a_d, ((0, 0), (0, DST_W - 8)))
                   + jnp.pad(m, ((0, 0), (8, DST_W - 16))))


def _tc2_body(acc_ref, ts1_ref, td1_ref, b1_ref, w2p_ref, as2_ref, ad2_ref,
              r8_ref, ts2_ref, td2_ref):
    acc = acc_ref[0] + acc_ref[1]
    h1 = ts1_ref[:, 0:64]
    a_s1 = ts1_ref[:, 64:72]
    a_d1 = td1_ref[:, 0:8]
    m1 = td1_ref[:, 8:16]
    ex = jnp.exp(_leaky(a_s1 + a_d1) - m1)
    s = acc[:, 64:72] + ex
    inv = 1.0 / (s + 1e-16)
    r8 = r8_ref[...]
    msg = acc[:, 0:64] + h1 * jnp.dot(ex, r8, preferred_element_type=jnp.float32)
    out1 = msg * jnp.dot(inv, r8, preferred_element_type=jnp.float32) + b1_ref[...]
    x2 = jnp.where(out1 > 0, out1, jnp.exp(jnp.minimum(out1, 0.0)) - 1.0)
    h2p = jnp.dot(x2, w2p_ref[...], preferred_element_type=jnp.float32)
    a2s = jnp.sum(h2p * as2_ref[...], axis=1, keepdims=True)
    a2d = jnp.sum(h2p * ad2_ref[...], axis=1, keepdims=True)
    gmax2 = jnp.max(a2s)
    m2 = _leaky(a2d + gmax2)
    col = lax.broadcasted_iota(jnp.int32, (N, SRC_W2), 1)
    ts2_ref[...] = (jnp.where(col < 40, h2p, 0.0)
                    + jnp.where(col == 40, 1.0, 0.0)
                    + jnp.where(col == 41, a2s, 0.0))
    col16 = lax.broadcasted_iota(jnp.int32, (N, DST_W2), 1)
    td2_ref[...] = (jnp.where(col16 == 0, a2d, 0.0)
                    + jnp.where(col16 == 1, m2, 0.0))


def _tc3_body(acc2_ref, ts2_ref, td2_ref, b2_ref, as2_ref, ad2_ref, out_ref):
    acc = acc2_ref[0] + acc2_ref[1]
    col = lax.broadcasted_iota(jnp.int32, (N, SRC_W2), 1)
    h2 = jnp.where(col < 40, ts2_ref[...], 0.0)
    a2s = jnp.sum(h2 * as2_ref[...], axis=1, keepdims=True)
    td2 = td2_ref[...]
    col16 = lax.broadcasted_iota(jnp.int32, (N, DST_W2), 1)
    a2d = jnp.sum(jnp.where(col16 == 0, td2, 0.0), axis=1, keepdims=True)
    m2 = jnp.sum(jnp.where(col16 == 1, td2, 0.0), axis=1, keepdims=True)
    ex = jnp.exp(_leaky(a2s + a2d) - m2)
    s2 = jnp.sum(jnp.where(col == 40, acc, 0.0), axis=1, keepdims=True) + ex
    msg = acc[:, 0:40] + h2[:, 0:40] * ex
    out2 = msg / (s2 + 1e-16) + b2_ref[...]
    mx = jnp.max(out2, axis=1, keepdims=True)
    z = out2 - mx
    out_ref[...] = z - jnp.log(jnp.sum(jnp.exp(z), axis=1, keepdims=True))


def _lane_iota():
    return lax.iota(jnp.int32, 16)


def _permute(v, idx):
    dn = lax.GatherDimensionNumbers(offset_dims=(), collapsed_slice_dims=(0,),
                                    start_index_map=(0,))
    return lax.gather(v, idx[:, None], dn, slice_sizes=(1,),
                      mode=lax.GatherScatterMode.PROMISE_IN_BOUNDS)


def _zero_acc(zbuf, acc_sh, width, row0):
    def zrow(i, carry):
        for k in range(width // 16):
            zbuf[i, pl.ds(16 * k, 16)] = jnp.zeros((16,), jnp.float32)
        return carry
    lax.fori_loop(0, ZR, zrow, 0)
    for j in range(RPT // ZR):
        pltpu.sync_copy(zbuf, acc_sh.at[pl.ds(row0 + j * ZR, ZR)])


def _writeback(acc_sh, out_hbm, cid, row0):
    for j in range(RPT // ZR):
        pltpu.sync_copy(acc_sh.at[pl.ds(row0 + j * ZR, ZR)],
                        out_hbm.at[cid, pl.ds(row0 + j * ZR, ZR)])


def _edge1(srows, drows, mbuf, e, consts):
    hi = consts
    a = srows[e, pl.ds(64, 16)]
    va = drows[e, pl.ds(0, 16)]
    vb = drows[e, pl.ds(8, 16)]
    ex = jnp.exp(_leaky(a + va) - vb)
    for k in range(4):
        hk = srows[e, pl.ds(16 * k, 16)]
        exk = _permute(ex, 2 * k + hi)
        mbuf[e, pl.ds(16 * k, 16)] = hk * exk
    mbuf[e, pl.ds(64, 16)] = ex


def _edge2(srows, drows, mbuf, e, consts):
    c9, c0, c1 = consts
    a = srows[e, pl.ds(32, 16)]
    vd = drows[e, pl.ds(0, 16)]
    exb = jnp.exp(_leaky(_permute(a, c9) + _permute(vd, c0))
                  - _permute(vd, c1))
    for k in range(3):
        hk = srows[e, pl.ds(16 * k, 16)]
        mbuf[e, pl.ds(16 * k, 16)] = hk * exb


def _make_sc_body(edge_fn, make_consts, src_w):
    def body(ts_hbm, td_hbm, src_hbm, dst_hbm, out_hbm,
             src_all, dst_all, srows0, srows1, drows0, drows1, mbuf0, mbuf1,
             zbuf, acc_sh, ss0, ss1, sd0, sd1, sm0, sm1):
        cid = lax.axis_index("c")
        sid = lax.axis_index("s")
        wid = sid * NC + cid
        row0 = sid * RPT
        _zero_acc(zbuf, acc_sh, src_w, row0)
        pltpu.sync_copy(src_hbm.at[wid], src_all)
        pltpu.sync_copy(dst_hbm.at[wid], dst_all)
        plsc.subcore_barrier()

        consts = make_consts()
        srows = [srows0, srows1]
        drows = [drows0, drows1]
        mbufs = [mbuf0, mbuf1]
        sems_s = [ss0, ss1]
        sems_d = [sd0, sd1]
        sems_m = [sm0, sm1]

        def fetch(c, b):
            pltpu.async_copy(ts_hbm.at[src_all.at[c]], srows[b], sems_s[b])
            pltpu.async_copy(td_hbm.at[dst_all.at[c]], drows[b], sems_d[b])

        def wait(c, b):
            pltpu.make_async_copy(ts_hbm.at[src_all.at[c]], srows[b],
                                  sems_s[b]).wait()
            pltpu.make_async_copy(td_hbm.at[dst_all.at[c]], drows[b],
                                  sems_d[b]).wait()

        def wait_scatter(c, b):
            pltpu.make_async_copy(mbufs[b], acc_sh.at[dst_all.at[c]],
                                  sems_m[b]).wait()

        fetch(0, 0)

        def loop(g, carry):
            for b in range(2):
                c = 2 * g + b

                @pl.when(c < NCHUNK)
                def _():
                    @pl.when(c + 1 < NCHUNK)
                    def _():
                        fetch(c + 1, 1 - b)
                    wait(c, b)

                    @pl.when(c >= 2)
                    def _():
                        wait_scatter(c, b)

                    @plsc.parallel_loop(0, CH, unroll=8)
                    def _(e):
                        edge_fn(srows[b], drows[b], mbufs[b], e, consts)
                    pltpu.async_copy(mbufs[b], acc_sh.at[dst_all.at[c]],
                                     sems_m[b], add=True)
            return carry
        lax.fori_loop(0, (NCHUNK + 1) // 2, loop, 0)
        wait_scatter(NCHUNK - 1, (NCHUNK - 1) % 2)
        wait_scatter(NCHUNK - 2, (NCHUNK - 2) % 2)
        plsc.subcore_barrier()
        _writeback(acc_sh, out_hbm, cid, row0)
    return body


def _consts1():
    return jnp.where(_lane_iota() >= 8, 1, 0)


def _consts2():
    lanes = _lane_iota()
    return lanes * 0 + 9, lanes * 0, lanes * 0 + 1


_mesh = plsc.VectorSubcoreMesh(core_axis_name="c", subcore_axis_name="s")


def _make_sc(edge_fn, make_consts, src_w, dst_w):
    body = _make_sc_body(edge_fn, make_consts, src_w)
    return functools.partial(
        pl.kernel,
        out_type=jax.ShapeDtypeStruct((NC, N, src_w), jnp.float32),
        mesh=_mesh,
        scratch_types=[
            pltpu.VMEM((NCHUNK, CH), jnp.int32),
            pltpu.VMEM((NCHUNK, CH), jnp.int32),
            pltpu.VMEM((CH, src_w), jnp.float32),
            pltpu.VMEM((CH, src_w), jnp.float32),
            pltpu.VMEM((CH, dst_w), jnp.float32),
            pltpu.VMEM((CH, dst_w), jnp.float32),
            pltpu.VMEM((CH, src_w), jnp.float32),
            pltpu.VMEM((CH, src_w), jnp.float32),
            pltpu.VMEM((ZR, src_w), jnp.float32),
            pltpu.VMEM_SHARED((N, src_w), jnp.float32),
            pltpu.SemaphoreType.DMA,
            pltpu.SemaphoreType.DMA,
            pltpu.SemaphoreType.DMA,
            pltpu.SemaphoreType.DMA,
            pltpu.SemaphoreType.DMA,
            pltpu.SemaphoreType.DMA,
        ],
        compiler_params=pltpu.CompilerParams(use_tc_tiling_on_sc=False),
    )(body)


_sc1 = _make_sc(_edge1, _consts1, SRC_W, DST_W)
_sc2 = _make_sc(_edge2, _consts2, SRC_W2, DST_W2)


def kernel(x, edge_index, W1, att_src1, att_dst1, b1, W2, att_src2, att_dst2, b2):
    f32 = jnp.float32
    eye_h = jnp.eye(H, dtype=f32)
    As = (att_src1[:, :, None] * eye_h[:, None, :]).reshape(H * C1, H)
    Ad = (att_dst1[:, :, None] * eye_h[:, None, :]).reshape(H * C1, H)
    R8 = jnp.kron(eye_h, jnp.ones((1, C1), dtype=f32))
    W2p = jnp.pad(W2, ((0, 0), (0, SRC_W2 - NCLS)))
    as2 = jnp.pad(att_src2, ((0, 0), (0, SRC_W2 - NCLS)))
    ad2 = jnp.pad(att_dst2, ((0, 0), (0, SRC_W2 - NCLS)))
    b1r = b1.reshape(1, H * C1)
    b2r = b2.reshape(1, NCLS)
    src = edge_index[0].reshape(NW, NCHUNK, CH)
    dst = edge_index[1].reshape(NW, NCHUNK, CH)

    ts1, td1 = pl.pallas_call(
        _tc1_body,
        out_shape=[jax.ShapeDtypeStruct((N, SRC_W), f32),
                   jax.ShapeDtypeStruct((N, DST_W), f32)],
    )(x, W1, As, Ad)

    acc1 = _sc1(ts1, td1, src, dst)

    ts2, td2 = pl.pallas_call(
        _tc2_body,
        out_shape=[jax.ShapeDtypeStruct((N, SRC_W2), f32),
                   jax.ShapeDtypeStruct((N, DST_W2), f32)],
    )(acc1, ts1, td1, b1r, W2p, as2, ad2, R8)

    acc2 = _sc2(ts2, td2, src, dst)

    out = pl.pallas_call(
        _tc3_body,
        out_shape=jax.ShapeDtypeStruct((N, NCLS), f32),
    )(acc2, ts2, td2, b2r, as2, ad2)
    return out


# speedup vs baseline: 1.0310x; 1.0310x over previous
import functools

import numpy as np

import jax
import jax.numpy as jnp
from jax import lax
from jax.experimental import pallas as pl
from jax.experimental.pallas import tpu as pltpu
from jax.experimental.pallas import tpu_sc as plsc

N = 10000
E = 320000
D = 128
H = 8
C1 = 8
NCLS = 40

NC = 2
NS = 16
NW = NC * NS
EPW = E // NW
CH = 80
NCHUNK = EPW // CH
RPT = N // NS
ZR = 125

ACC_W1 = 80
ACC_W2 = 64

_NEG_SLOPE = 0.2

_PERM = [32 * (p // 32) + 16 * (p % 2) + (p % 32) // 2 for p in range(64)]
_PINV1 = np.zeros((64, 64), np.float32)
for _p, _c in enumerate(_PERM):
    _PINV1[_p, _c] = 1.0
_PINV2 = np.zeros((64, 48), np.float32)
for _p, _c in enumerate(_PERM):
    if _c < 48:
        _PINV2[_p, _c] = 1.0


def _leaky(t):
    return jnp.where(t >= 0, t, _NEG_SLOPE * t)


def _tc1_body(x_ref, w1p_ref, asp_ref, adp_ref, tsh_ref, tsa_ref, td_ref):
    hperm = jnp.dot(x_ref[...], w1p_ref[...], preferred_element_type=jnp.float32)
    a_s = jnp.dot(hperm, asp_ref[...], preferred_element_type=jnp.float32)
    a_d = jnp.dot(hperm, adp_ref[...], preferred_element_type=jnp.float32)
    gmax = jnp.max(a_s, axis=0, keepdims=True)
    m = _leaky(a_d + gmax)
    tsh_ref[...] = hperm.astype(jnp.bfloat16)
    tsa_ref[...] = jnp.pad(a_s, ((0, 0), (0, 8)))
    td_ref[...] = jnp.pad(a_d, ((0, 0), (0, 8))) + jnp.pad(m, ((0, 0), (8, 0)))


def _tc2_body(acc_ref, tsh1_ref, tsa1_ref, td1_ref, b1_ref, w2p_ref, as2_ref,
              ad2_ref, r8_ref, pinv1_ref, tsh2_ref, tsa2_ref, td2_ref):
    acc = acc_ref[0] + acc_ref[1]
    h1 = jnp.dot(tsh1_ref[...].astype(jnp.float32), pinv1_ref[...],
                 preferred_element_type=jnp.float32)
    a_s1 = tsa1_ref[:, 0:8]
    a_d1 = td1_ref[:, 0:8]
    m1 = td1_ref[:, 8:16]
    ex = jnp.exp(_leaky(a_s1 + a_d1) - m1)
    s = acc[:, 64:72] + ex
    inv = 1.0 / (s + 1e-16)
    r8 = r8_ref[...]
    msg = acc[:, 0:64] + h1 * jnp.dot(ex, r8, preferred_element_type=jnp.float32)
    out1 = msg * jnp.dot(inv, r8, preferred_element_type=jnp.float32) + b1_ref[...]
    x2 = jnp.where(out1 > 0, out1, jnp.exp(jnp.minimum(out1, 0.0)) - 1.0)
    h2perm = jnp.dot(x2, w2p_ref[...], preferred_element_type=jnp.float32)
    a2s = jnp.sum(h2perm * as2_ref[...], axis=1, keepdims=True)
    a2d = jnp.sum(h2perm * ad2_ref[...], axis=1, keepdims=True)
    gmax2 = jnp.max(a2s)
    m2 = _leaky(a2d + gmax2)
    tsh2_ref[...] = h2perm.astype(jnp.bfloat16)
    col16 = lax.broadcasted_iota(jnp.int32, (N, 16), 1)
    tsa2_ref[...] = (jnp.where(col16 == 0, 1.0, 0.0)
                     + jnp.where(col16 == 1, a2s, 0.0))
    td2_ref[...] = (jnp.where(col16 == 0, a2d, 0.0)
                    + jnp.where(col16 == 1, m2, 0.0))


def _tc3_body(acc2_ref, tsh2_ref, tsa2_ref, td2_ref, b2_ref, pinv2_ref,
              out_ref):
    acc = acc2_ref[0] + acc2_ref[1]
    h2 = jnp.dot(tsh2_ref[...].astype(jnp.float32), pinv2_ref[...],
                 preferred_element_type=jnp.float32)
    col16 = lax.broadcasted_iota(jnp.int32, (N, 16), 1)
    tsa2 = tsa2_ref[...]
    td2 = td2_ref[...]
    a2s = jnp.sum(jnp.where(col16 == 1, tsa2, 0.0), axis=1, keepdims=True)
    a2d = jnp.sum(jnp.where(col16 == 0, td2, 0.0), axis=1, keepdims=True)
    m2 = jnp.sum(jnp.where(col16 == 1, td2, 0.0), axis=1, keepdims=True)
    ex = jnp.exp(_leaky(a2s + a2d) - m2)
    col64 = lax.broadcasted_iota(jnp.int32, (N, ACC_W2), 1)
    s2 = jnp.sum(jnp.where(col64 == 48, acc, 0.0), axis=1, keepdims=True) + ex
    msg = acc[:, 0:40] + h2[:, 0:40] * ex
    out2 = msg / (s2 + 1e-16) + b2_ref[...]
    mx = jnp.max(out2, axis=1, keepdims=True)
    z = out2 - mx
    out_ref[...] = z - jnp.log(jnp.sum(jnp.exp(z), axis=1, keepdims=True))


def _lane_iota():
    return lax.iota(jnp.int32, 16)


def _permute(v, idx):
    dn = lax.GatherDimensionNumbers(offset_dims=(), collapsed_slice_dims=(0,),
                                    start_index_map=(0,))
    return lax.gather(v, idx[:, None], dn, slice_sizes=(1,),
                      mode=lax.GatherScatterMode.PROMISE_IN_BOUNDS)


def _unpack2(hrows, e, off):
    u = hrows[e, pl.ds(off, 32)]
    return plsc.unpack(u, format=plsc.PackFormat.INTERLEAVED,
                       preferred_element_type=jnp.float32)


def _zero_acc(zbuf, acc_sh, width, row0):
    def zrow(i, carry):
        for k in range(width // 16):
            zbuf[i, pl.ds(16 * k, 16)] = jnp.zeros((16,), jnp.float32)
        return carry
    lax.fori_loop(0, ZR, zrow, 0)
    for j in range(RPT // ZR):
        pltpu.sync_copy(zbuf, acc_sh.at[pl.ds(row0 + j * ZR, ZR)])


def _writeback(acc_sh, out_hbm, cid, row0):
    for j in range(RPT // ZR):
        pltpu.sync_copy(acc_sh.at[pl.ds(row0 + j * ZR, ZR)],
                        out_hbm.at[cid, pl.ds(row0 + j * ZR, ZR)])


def _edge1(hrows, arows, drows, mbuf, e, consts):
    hi, hi8 = consts
    a = arows[e, pl.ds(0, 16)]
    vd = drows[e, pl.ds(0, 16)]
    vb = _permute(vd, hi8)
    ex = jnp.exp(_leaky(a + vd) - vb)
    u0a, u0b = _unpack2(hrows, e, 0)
    u1a, u1b = _unpack2(hrows, e, 32)
    for k, u in enumerate((u0a, u0b, u1a, u1b)):
        mbuf[e, pl.ds(16 * k, 16)] = u * _permute(ex, 2 * k + hi)
    mbuf[e, pl.ds(64, 16)] = ex


def _edge2(hrows, arows, drows, mbuf, e, consts):
    c0, c1 = consts
    va = arows[e, pl.ds(0, 16)]
    vd = drows[e, pl.ds(0, 16)]
    exb = jnp.exp(_leaky(_permute(va, c1) + _permute(vd, c0))
                  - _permute(vd, c1))
    u0a, u0b = _unpack2(hrows, e, 0)
    u1a, _ = _unpack2(hrows, e, 32)
    mbuf[e, pl.ds(0, 16)] = u0a * exb
    mbuf[e, pl.ds(16, 16)] = u0b * exb
    mbuf[e, pl.ds(32, 16)] = u1a * exb
    mbuf[e, pl.ds(48, 16)] = va * exb


def _consts1():
    lanes = _lane_iota()
    return jnp.where(lanes >= 8, 1, 0), (lanes % 8) + 8


def _consts2():
    lanes = _lane_iota()
    return lanes * 0, lanes * 0 + 1


def _make_sc_body(edge_fn, make_consts, acc_w):
    def body(tsh_hbm, tsa_hbm, td_hbm, src_hbm, dst_hbm, out_hbm,
             src_all, dst_all, hrows0, hrows1, arows0, arows1, drows0, drows1,
             mbuf0, mbuf1, zbuf, acc_sh,
             sh0, sh1, sa0, sa1, sd0, sd1, sm0, sm1):
        cid = lax.axis_index("c")
        sid = lax.axis_index("s")
        wid = sid * NC + cid
        row0 = sid * RPT
        _zero_acc(zbuf, acc_sh, acc_w, row0)
        pltpu.sync_copy(src_hbm.at[wid], src_all)
        pltpu.sync_copy(dst_hbm.at[wid], dst_all)
        plsc.subcore_barrier()

        consts = make_consts()
        hrows = [hrows0, hrows1]
        arows = [arows0, arows1]
        drows = [drows0, drows1]
        mbufs = [mbuf0, mbuf1]
        sems_h = [sh0, sh1]
        sems_a = [sa0, sa1]
        sems_d = [sd0, sd1]
        sems_m = [sm0, sm1]

        def fetch(c, b):
            pltpu.async_copy(tsh_hbm.at[src_all.at[c]], hrows[b], sems_h[b])
            pltpu.async_copy(tsa_hbm.at[src_all.at[c]], arows[b], sems_a[b])
            pltpu.async_copy(td_hbm.at[dst_all.at[c]], drows[b], sems_d[b])

        def wait(c, b):
            pltpu.make_async_copy(tsh_hbm.at[src_all.at[c]], hrows[b],
                                  sems_h[b]).wait()
            pltpu.make_async_copy(tsa_hbm.at[src_all.at[c]], arows[b],
                                  sems_a[b]).wait()
            pltpu.make_async_copy(td_hbm.at[dst_all.at[c]], drows[b],
                                  sems_d[b]).wait()

        def wait_scatter(c, b):
            pltpu.make_async_copy(mbufs[b], acc_sh.at[dst_all.at[c]],
                                  sems_m[b]).wait()

        fetch(0, 0)

        def loop(g, carry):
            for b in range(2):
                c = 2 * g + b

                @pl.when(c < NCHUNK)
                def _():
                    @pl.when(c + 1 < NCHUNK)
                    def _():
                        fetch(c + 1, 1 - b)
                    wait(c, b)

                    @pl.when(c >= 2)
                    def _():
                        wait_scatter(c, b)

                    @plsc.parallel_loop(0, CH, unroll=4)
                    def _(e):
                        edge_fn(hrows[b], arows[b], drows[b], mbufs[b], e,
                                consts)
                    pltpu.async_copy(mbufs[b], acc_sh.at[dst_all.at[c]],
                                     sems_m[b], add=True)
            return carry
        lax.fori_loop(0, (NCHUNK + 1) // 2, loop, 0)
        wait_scatter(NCHUNK - 1, (NCHUNK - 1) % 2)
        wait_scatter(NCHUNK - 2, (NCHUNK - 2) % 2)
        plsc.subcore_barrier()
        _writeback(acc_sh, out_hbm, cid, row0)
    return body


_mesh = plsc.VectorSubcoreMesh(core_axis_name="c", subcore_axis_name="s")


def _make_sc(edge_fn, make_consts, acc_w):
    body = _make_sc_body(edge_fn, make_consts, acc_w)
    return functools.partial(
        pl.kernel,
        out_type=jax.ShapeDtypeStruct((NC, N, acc_w), jnp.float32),
        mesh=_mesh,
        scratch_types=[
            pltpu.VMEM((NCHUNK, CH), jnp.int32),
            pltpu.VMEM((NCHUNK, CH), jnp.int32),
            pltpu.VMEM((CH, 64), jnp.bfloat16),
            pltpu.VMEM((CH, 64), jnp.bfloat16),
            pltpu.VMEM((CH, 16), jnp.float32),
            pltpu.VMEM((CH, 16), jnp.float32),
            pltpu.VMEM((CH, 16), jnp.float32),
            pltpu.VMEM((CH, 16), jnp.float32),
            pltpu.VMEM((CH, acc_w), jnp.float32),
            pltpu.VMEM((CH, acc_w), jnp.float32),
            pltpu.VMEM((ZR, acc_w), jnp.float32),
            pltpu.VMEM_SHARED((N, acc_w), jnp.float32),
            pltpu.SemaphoreType.DMA,
            pltpu.SemaphoreType.DMA,
            pltpu.SemaphoreType.DMA,
            pltpu.SemaphoreType.DMA,
            pltpu.SemaphoreType.DMA,
            pltpu.SemaphoreType.DMA,
            pltpu.SemaphoreType.DMA,
            pltpu.SemaphoreType.DMA,
        ],
        compiler_params=pltpu.CompilerParams(use_tc_tiling_on_sc=False,
                                             needs_layout_passes=False),
    )(body)


_sc1 = _make_sc(_edge1, _consts1, ACC_W1)
_sc2 = _make_sc(_edge2, _consts2, ACC_W2)


def kernel(x, edge_index, W1, att_src1, att_dst1, b1, W2, att_src2, att_dst2, b2):
    f32 = jnp.float32
    perm = jnp.array(_PERM, dtype=jnp.int32)
    eye_h = jnp.eye(H, dtype=f32)
    As = (att_src1[:, :, None] * eye_h[:, None, :]).reshape(H * C1, H)
    Ad = (att_dst1[:, :, None] * eye_h[:, None, :]).reshape(H * C1, H)
    W1p = W1[:, perm]
    Asp = As[perm, :]
    Adp = Ad[perm, :]
    R8 = jnp.kron(eye_h, jnp.ones((1, C1), dtype=f32))
    W2pad = jnp.pad(W2, ((0, 0), (0, 64 - NCLS)))
    W2p = W2pad[:, perm]
    as2 = jnp.pad(att_src2, ((0, 0), (0, 64 - NCLS)))[:, perm]
    ad2 = jnp.pad(att_dst2, ((0, 0), (0, 64 - NCLS)))[:, perm]
    pinv1 = jnp.asarray(_PINV1)
    pinv2 = jnp.asarray(_PINV2)
    b1r = b1.reshape(1, H * C1)
    b2r = b2.reshape(1, NCLS)
    src = edge_index[0].reshape(NW, NCHUNK, CH)
    dst = edge_index[1].reshape(NW, NCHUNK, CH)

    tsh1, tsa1, td1 = pl.pallas_call(
        _tc1_body,
        out_shape=[jax.ShapeDtypeStruct((N, 64), jnp.bfloat16),
                   jax.ShapeDtypeStruct((N, 16), f32),
                   jax.ShapeDtypeStruct((N, 16), f32)],
    )(x, W1p, Asp, Adp)

    acc1 = _sc1(tsh1, tsa1, td1, src, dst)

    tsh2, tsa2, td2 = pl.pallas_call(
        _tc2_body,
        out_shape=[jax.ShapeDtypeStruct((N, 64), jnp.bfloat16),
                   jax.ShapeDtypeStruct((N, 16), f32),
                   jax.ShapeDtypeStruct((N, 16), f32)],
    )(acc1, tsh1, tsa1, td1, b1r, W2p, as2, ad2, R8, pinv1)

    acc2 = _sc2(tsh2, tsa2, td2, src, dst)

    out = pl.pallas_call(
        _tc3_body,
        out_shape=jax.ShapeDtypeStruct((N, NCLS), f32),
    )(acc2, tsh2, tsa2, td2, b2r, pinv2)
    return out


# speedup vs baseline: 1.0471x; 1.0156x over previous
import functools

import numpy as np

import jax
import jax.numpy as jnp
from jax import lax
from jax.experimental import pallas as pl
from jax.experimental.pallas import tpu as pltpu
from jax.experimental.pallas import tpu_sc as plsc

N = 10000
E = 320000
D = 128
H = 8
C1 = 8
NCLS = 40

NC = 2
NS = 16
NW = NC * NS
EPW = E // NW
CH = 80
NCHUNK = EPW // CH
RPT = N // NS
ZR = 125

ACC_W1 = 80
ACC_W2 = 64

_NEG_SLOPE = 0.2

_PERM = [32 * (p // 32) + 16 * (p % 2) + (p % 32) // 2 for p in range(64)]
_PINV1 = np.zeros((64, 64), np.float32)
for _p, _c in enumerate(_PERM):
    _PINV1[_p, _c] = 1.0
_PINV2 = np.zeros((64, 48), np.float32)
for _p, _c in enumerate(_PERM):
    if _c < 48:
        _PINV2[_p, _c] = 1.0


def _leaky(t):
    return jnp.where(t >= 0, t, _NEG_SLOPE * t)


def _tc1_body(x_ref, w1p_ref, asp_ref, adp_ref, tsh_ref, tsa_ref, td_ref):
    hperm = jnp.dot(x_ref[...], w1p_ref[...], preferred_element_type=jnp.float32)
    a_s = jnp.dot(hperm, asp_ref[...], preferred_element_type=jnp.float32)
    a_d = jnp.dot(hperm, adp_ref[...], preferred_element_type=jnp.float32)
    gmax = jnp.max(a_s, axis=0, keepdims=True)
    m = _leaky(a_d + gmax)
    tsh_ref[...] = hperm.astype(jnp.bfloat16)
    tsa_ref[...] = jnp.pad(a_s, ((0, 0), (0, 8)))
    td_ref[...] = jnp.pad(a_d, ((0, 0), (0, 8))) + jnp.pad(m, ((0, 0), (8, 0)))


def _tc2_body(acc_ref, tsh1_ref, tsa1_ref, td1_ref, b1_ref, w2p_ref, as2_ref,
              ad2_ref, r8_ref, pinv1_ref, tsh2_ref, tsa2_ref, td2_ref):
    acc = acc_ref[0] + acc_ref[1]
    h1 = jnp.dot(tsh1_ref[...].astype(jnp.float32), pinv1_ref[...],
                 preferred_element_type=jnp.float32)
    a_s1 = tsa1_ref[:, 0:8]
    a_d1 = td1_ref[:, 0:8]
    m1 = td1_ref[:, 8:16]
    ex = jnp.exp(_leaky(a_s1 + a_d1) - m1)
    s = acc[:, 64:72] + ex
    inv = 1.0 / (s + 1e-16)
    r8 = r8_ref[...]
    msg = acc[:, 0:64] + h1 * jnp.dot(ex, r8, preferred_element_type=jnp.float32)
    out1 = msg * jnp.dot(inv, r8, preferred_element_type=jnp.float32) + b1_ref[...]
    x2 = jnp.where(out1 > 0, out1, jnp.exp(jnp.minimum(out1, 0.0)) - 1.0)
    h2perm = jnp.dot(x2, w2p_ref[...], preferred_element_type=jnp.float32)
    a2s = jnp.sum(h2perm * as2_ref[...], axis=1, keepdims=True)
    a2d = jnp.sum(h2perm * ad2_ref[...], axis=1, keepdims=True)
    gmax2 = jnp.max(a2s)
    m2 = _leaky(a2d + gmax2)
    tsh2_ref[...] = h2perm.astype(jnp.bfloat16)
    col16 = lax.broadcasted_iota(jnp.int32, (N, 16), 1)
    tsa2_ref[...] = (jnp.where(col16 == 0, 1.0, 0.0)
                     + jnp.where(col16 == 1, a2s, 0.0))
    td2_ref[...] = (jnp.where(col16 == 0, a2d, 0.0)
                    + jnp.where(col16 == 1, m2, 0.0))


def _tc3_body(acc2_ref, tsh2_ref, tsa2_ref, td2_ref, b2_ref, pinv2_ref,
              out_ref):
    acc = acc2_ref[0] + acc2_ref[1]
    h2 = jnp.dot(tsh2_ref[...].astype(jnp.float32), pinv2_ref[...],
                 preferred_element_type=jnp.float32)
    col16 = lax.broadcasted_iota(jnp.int32, (N, 16), 1)
    tsa2 = tsa2_ref[...]
    td2 = td2_ref[...]
    a2s = jnp.sum(jnp.where(col16 == 1, tsa2, 0.0), axis=1, keepdims=True)
    a2d = jnp.sum(jnp.where(col16 == 0, td2, 0.0), axis=1, keepdims=True)
    m2 = jnp.sum(jnp.where(col16 == 1, td2, 0.0), axis=1, keepdims=True)
    ex = jnp.exp(_leaky(a2s + a2d) - m2)
    col64 = lax.broadcasted_iota(jnp.int32, (N, ACC_W2), 1)
    s2 = jnp.sum(jnp.where(col64 == 48, acc, 0.0), axis=1, keepdims=True) + ex
    msg = acc[:, 0:40] + h2[:, 0:40] * ex
    out2 = msg / (s2 + 1e-16) + b2_ref[...]
    mx = jnp.max(out2, axis=1, keepdims=True)
    z = out2 - mx
    out_ref[...] = z - jnp.log(jnp.sum(jnp.exp(z), axis=1, keepdims=True))


def _lane_iota():
    return lax.iota(jnp.int32, 16)


def _permute(v, idx):
    dn = lax.GatherDimensionNumbers(offset_dims=(), collapsed_slice_dims=(0,),
                                    start_index_map=(0,))
    return lax.gather(v, idx[:, None], dn, slice_sizes=(1,),
                      mode=lax.GatherScatterMode.PROMISE_IN_BOUNDS)


def _unpack2(hrows, e, off):
    u = hrows[e, pl.ds(off, 32)]
    return plsc.unpack(u, format=plsc.PackFormat.INTERLEAVED,
                       preferred_element_type=jnp.float32)


def _zero_acc(zbuf, acc_sh, width, row0):
    def zrow(i, carry):
        for k in range(width // 16):
            zbuf[i, pl.ds(16 * k, 16)] = jnp.zeros((16,), jnp.float32)
        return carry
    lax.fori_loop(0, ZR, zrow, 0)
    for j in range(RPT // ZR):
        pltpu.sync_copy(zbuf, acc_sh.at[pl.ds(row0 + j * ZR, ZR)])


def _writeback(acc_sh, out_hbm, cid, row0):
    for j in range(RPT // ZR):
        pltpu.sync_copy(acc_sh.at[pl.ds(row0 + j * ZR, ZR)],
                        out_hbm.at[cid, pl.ds(row0 + j * ZR, ZR)])


def _edge1(hrows, arows, drows, mbuf, e, consts):
    hi, hi8 = consts
    a = arows[e, pl.ds(0, 16)]
    vd = drows[e, pl.ds(0, 16)]
    vb = _permute(vd, hi8)
    ex = jnp.exp(_leaky(a + vd) - vb)
    u0a, u0b = _unpack2(hrows, e, 0)
    u1a, u1b = _unpack2(hrows, e, 32)
    for k, u in enumerate((u0a, u0b, u1a, u1b)):
        mbuf[e, pl.ds(16 * k, 16)] = u * _permute(ex, 2 * k + hi)
    mbuf[e, pl.ds(64, 16)] = ex


def _edge2(hrows, arows, drows, mbuf, e, consts):
    c0, c1 = consts
    va = arows[e, pl.ds(0, 16)]
    vd = drows[e, pl.ds(0, 16)]
    exb = jnp.exp(_leaky(_permute(va, c1) + _permute(vd, c0))
                  - _permute(vd, c1))
    u0a, u0b = _unpack2(hrows, e, 0)
    u1a, _ = _unpack2(hrows, e, 32)
    mbuf[e, pl.ds(0, 16)] = u0a * exb
    mbuf[e, pl.ds(16, 16)] = u0b * exb
    mbuf[e, pl.ds(32, 16)] = u1a * exb
    mbuf[e, pl.ds(48, 16)] = va * exb


def _consts1():
    lanes = _lane_iota()
    return jnp.where(lanes >= 8, 1, 0), (lanes % 8) + 8


def _consts2():
    lanes = _lane_iota()
    return lanes * 0, lanes * 0 + 1


def _make_sc_body(edge_fn, make_consts, acc_w):
    def body(tsh_hbm, tsa_hbm, td_hbm, src_hbm, dst_hbm, out_hbm,
             src_all, dst_all, hrows0, hrows1, arows0, arows1, drows0, drows1,
             mbuf0, mbuf1, zbuf, acc_sh, tsh_sp,
             sh0, sh1, sa0, sa1, sd0, sd1, sm0, sm1):
        cid = lax.axis_index("c")
        sid = lax.axis_index("s")
        wid = sid * NC + cid
        row0 = sid * RPT
        _zero_acc(zbuf, acc_sh, acc_w, row0)
        pltpu.sync_copy(src_hbm.at[wid], src_all)
        pltpu.sync_copy(dst_hbm.at[wid], dst_all)
        band = pl.ds(row0, RPT)
        pltpu.sync_copy(tsh_hbm.at[band], tsh_sp.at[band])
        plsc.subcore_barrier()

        consts = make_consts()
        hrows = [hrows0, hrows1]
        arows = [arows0, arows1]
        drows = [drows0, drows1]
        mbufs = [mbuf0, mbuf1]
        sems_h = [sh0, sh1]
        sems_a = [sa0, sa1]
        sems_d = [sd0, sd1]
        sems_m = [sm0, sm1]

        def fetch(c, b):
            pltpu.async_copy(tsh_sp.at[src_all.at[c]], hrows[b], sems_h[b])
            pltpu.async_copy(tsa_hbm.at[src_all.at[c]], arows[b], sems_a[b])
            pltpu.async_copy(td_hbm.at[dst_all.at[c]], drows[b], sems_d[b])

        def wait(c, b):
            pltpu.make_async_copy(tsh_sp.at[src_all.at[c]], hrows[b],
                                  sems_h[b]).wait()
            pltpu.make_async_copy(tsa_hbm.at[src_all.at[c]], arows[b],
                                  sems_a[b]).wait()
            pltpu.make_async_copy(td_hbm.at[dst_all.at[c]], drows[b],
                                  sems_d[b]).wait()

        def wait_scatter(c, b):
            pltpu.make_async_copy(mbufs[b], acc_sh.at[dst_all.at[c]],
                                  sems_m[b]).wait()

        fetch(0, 0)

        def loop(g, carry):
            for b in range(2):
                c = 2 * g + b

                @pl.when(c < NCHUNK)
                def _():
                    @pl.when(c + 1 < NCHUNK)
                    def _():
                        fetch(c + 1, 1 - b)
                    wait(c, b)

                    @pl.when(c >= 2)
                    def _():
                        wait_scatter(c, b)

                    @plsc.parallel_loop(0, CH, unroll=4)
                    def _(e):
                        edge_fn(hrows[b], arows[b], drows[b], mbufs[b], e,
                                consts)
                    pltpu.async_copy(mbufs[b], acc_sh.at[dst_all.at[c]],
                                     sems_m[b], add=True)
            return carry
        lax.fori_loop(0, (NCHUNK + 1) // 2, loop, 0)
        wait_scatter(NCHUNK - 1, (NCHUNK - 1) % 2)
        wait_scatter(NCHUNK - 2, (NCHUNK - 2) % 2)
        plsc.subcore_barrier()
        _writeback(acc_sh, out_hbm, cid, row0)
    return body


_mesh = plsc.VectorSubcoreMesh(core_axis_name="c", subcore_axis_name="s")


def _make_sc(edge_fn, make_consts, acc_w):
    body = _make_sc_body(edge_fn, make_consts, acc_w)
    return functools.partial(
        pl.kernel,
        out_type=jax.ShapeDtypeStruct((NC, N, acc_w), jnp.float32),
        mesh=_mesh,
        scratch_types=[
            pltpu.VMEM((NCHUNK, CH), jnp.int32),
            pltpu.VMEM((NCHUNK, CH), jnp.int32),
            pltpu.VMEM((CH, 64), jnp.bfloat16),
            pltpu.VMEM((CH, 64), jnp.bfloat16),
            pltpu.VMEM((CH, 16), jnp.float32),
            pltpu.VMEM((CH, 16), jnp.float32),
            pltpu.VMEM((CH, 16), jnp.float32),
            pltpu.VMEM((CH, 16), jnp.float32),
            pltpu.VMEM((CH, acc_w), jnp.float32),
            pltpu.VMEM((CH, acc_w), jnp.float32),
            pltpu.VMEM((ZR, acc_w), jnp.float32),
            pltpu.VMEM_SHARED((N, acc_w), jnp.float32),
            pltpu.VMEM_SHARED((N, 64), jnp.bfloat16),
            pltpu.SemaphoreType.DMA,
            pltpu.SemaphoreType.DMA,
            pltpu.SemaphoreType.DMA,
            pltpu.SemaphoreType.DMA,
            pltpu.SemaphoreType.DMA,
            pltpu.SemaphoreType.DMA,
            pltpu.SemaphoreType.DMA,
            pltpu.SemaphoreType.DMA,
        ],
        compiler_params=pltpu.CompilerParams(use_tc_tiling_on_sc=False,
                                             needs_layout_passes=False),
    )(body)


_sc1 = _make_sc(_edge1, _consts1, ACC_W1)
_sc2 = _make_sc(_edge2, _consts2, ACC_W2)


def kernel(x, edge_index, W1, att_src1, att_dst1, b1, W2, att_src2, att_dst2, b2):
    f32 = jnp.float32
    perm = jnp.array(_PERM, dtype=jnp.int32)
    eye_h = jnp.eye(H, dtype=f32)
    As = (att_src1[:, :, None] * eye_h[:, None, :]).reshape(H * C1, H)
    Ad = (att_dst1[:, :, None] * eye_h[:, None, :]).reshape(H * C1, H)
    W1p = W1[:, perm]
    Asp = As[perm, :]
    Adp = Ad[perm, :]
    R8 = jnp.kron(eye_h, jnp.ones((1, C1), dtype=f32))
    W2pad = jnp.pad(W2, ((0, 0), (0, 64 - NCLS)))
    W2p = W2pad[:, perm]
    as2 = jnp.pad(att_src2, ((0, 0), (0, 64 - NCLS)))[:, perm]
    ad2 = jnp.pad(att_dst2, ((0, 0), (0, 64 - NCLS)))[:, perm]
    pinv1 = jnp.asarray(_PINV1)
    pinv2 = jnp.asarray(_PINV2)
    b1r = b1.reshape(1, H * C1)
    b2r = b2.reshape(1, NCLS)
    src = edge_index[0].reshape(NW, NCHUNK, CH)
    dst = edge_index[1].reshape(NW, NCHUNK, CH)

    tsh1, tsa1, td1 = pl.pallas_call(
        _tc1_body,
        out_shape=[jax.ShapeDtypeStruct((N, 64), jnp.bfloat16),
                   jax.ShapeDtypeStruct((N, 16), f32),
                   jax.ShapeDtypeStruct((N, 16), f32)],
    )(x, W1p, Asp, Adp)

    acc1 = _sc1(tsh1, tsa1, td1, src, dst)

    tsh2, tsa2, td2 = pl.pallas_call(
        _tc2_body,
        out_shape=[jax.ShapeDtypeStruct((N, 64), jnp.bfloat16),
                   jax.ShapeDtypeStruct((N, 16), f32),
                   jax.ShapeDtypeStruct((N, 16), f32)],
    )(acc1, tsh1, tsa1, td1, b1r, W2p, as2, ad2, R8, pinv1)

    acc2 = _sc2(tsh2, tsa2, td2, src, dst)

    out = pl.pallas_call(
        _tc3_body,
        out_shape=jax.ShapeDtypeStruct((N, NCLS), f32),
    )(acc2, tsh2, tsa2, td2, b2r, pinv2)
    return out


# speedup vs baseline: 1.1038x; 1.0541x over previous
import functools

import numpy as np

import jax
import jax.numpy as jnp
from jax import lax
from jax.experimental import pallas as pl
from jax.experimental.pallas import tpu as pltpu
from jax.experimental.pallas import tpu_sc as plsc

N = 10000
E = 320000
D = 128
H = 8
C1 = 8
NCLS = 40

NC = 2
NS = 16
NW = NC * NS
EPW = E // NW
CH = 80
NCHUNK = EPW // CH
RPT = N // NS
ZR = 125

ACC_W1 = 80
ACC_W2 = 64

_NEG_SLOPE = 0.2

_PERM = [32 * (p // 32) + 16 * (p % 2) + (p % 32) // 2 for p in range(64)]
_PINV1 = np.zeros((64, 64), np.float32)
for _p, _c in enumerate(_PERM):
    _PINV1[_p, _c] = 1.0
_PINV2 = np.zeros((64, 48), np.float32)
for _p, _c in enumerate(_PERM):
    if _c < 48:
        _PINV2[_p, _c] = 1.0


def _leaky(t):
    return jnp.where(t >= 0, t, _NEG_SLOPE * t)


def _tc1_body(x_ref, w1p_ref, asp_ref, adp_ref, tsh_ref, tsa_ref, td_ref):
    hperm = jnp.dot(x_ref[...], w1p_ref[...], preferred_element_type=jnp.float32)
    a_s = jnp.dot(hperm, asp_ref[...], preferred_element_type=jnp.float32)
    a_d = jnp.dot(hperm, adp_ref[...], preferred_element_type=jnp.float32)
    gmax = jnp.max(a_s, axis=0, keepdims=True)
    m = _leaky(a_d + gmax)
    tsh_ref[...] = hperm.astype(jnp.bfloat16)
    tsa_ref[...] = jnp.pad(a_s, ((0, 0), (0, 8)))
    td_ref[...] = jnp.pad(a_d, ((0, 0), (0, 8))) + jnp.pad(m, ((0, 0), (8, 0)))


def _tc2_body(acc_ref, tsh1_ref, tsa1_ref, td1_ref, b1_ref, w2p_ref, as2_ref,
              ad2_ref, r8_ref, pinv1_ref, tsh2_ref, tsa2_ref, td2_ref):
    acc = acc_ref[0] + acc_ref[1]
    h1 = jnp.dot(tsh1_ref[...].astype(jnp.float32), pinv1_ref[...],
                 preferred_element_type=jnp.float32)
    a_s1 = tsa1_ref[:, 0:8]
    a_d1 = td1_ref[:, 0:8]
    m1 = td1_ref[:, 8:16]
    ex = jnp.exp(_leaky(a_s1 + a_d1) - m1)
    s = acc[:, 64:72] + ex
    inv = 1.0 / (s + 1e-16)
    r8 = r8_ref[...]
    msg = acc[:, 0:64] + h1 * jnp.dot(ex, r8, preferred_element_type=jnp.float32)
    out1 = msg * jnp.dot(inv, r8, preferred_element_type=jnp.float32) + b1_ref[...]
    x2 = jnp.where(out1 > 0, out1, jnp.exp(jnp.minimum(out1, 0.0)) - 1.0)
    h2perm = jnp.dot(x2, w2p_ref[...], preferred_element_type=jnp.float32)
    a2s = jnp.sum(h2perm * as2_ref[...], axis=1, keepdims=True)
    a2d = jnp.sum(h2perm * ad2_ref[...], axis=1, keepdims=True)
    gmax2 = jnp.max(a2s)
    m2 = _leaky(a2d + gmax2)
    tsh2_ref[...] = h2perm.astype(jnp.bfloat16)
    col16 = lax.broadcasted_iota(jnp.int32, (N, 16), 1)
    tsa2_ref[...] = (jnp.where(col16 == 0, 1.0, 0.0)
                     + jnp.where(col16 == 1, a2s, 0.0))
    td2_ref[...] = (jnp.where(col16 == 0, a2d, 0.0)
                    + jnp.where(col16 == 1, m2, 0.0))


def _tc3_body(acc2_ref, tsh2_ref, tsa2_ref, td2_ref, b2_ref, pinv2_ref,
              out_ref):
    acc = acc2_ref[0] + acc2_ref[1]
    h2 = jnp.dot(tsh2_ref[...].astype(jnp.float32), pinv2_ref[...],
                 preferred_element_type=jnp.float32)
    col16 = lax.broadcasted_iota(jnp.int32, (N, 16), 1)
    tsa2 = tsa2_ref[...]
    td2 = td2_ref[...]
    a2s = jnp.sum(jnp.where(col16 == 1, tsa2, 0.0), axis=1, keepdims=True)
    a2d = jnp.sum(jnp.where(col16 == 0, td2, 0.0), axis=1, keepdims=True)
    m2 = jnp.sum(jnp.where(col16 == 1, td2, 0.0), axis=1, keepdims=True)
    ex = jnp.exp(_leaky(a2s + a2d) - m2)
    col64 = lax.broadcasted_iota(jnp.int32, (N, ACC_W2), 1)
    s2 = jnp.sum(jnp.where(col64 == 48, acc, 0.0), axis=1, keepdims=True) + ex
    msg = acc[:, 0:40] + h2[:, 0:40] * ex
    out2 = msg / (s2 + 1e-16) + b2_ref[...]
    mx = jnp.max(out2, axis=1, keepdims=True)
    z = out2 - mx
    out_ref[...] = z - jnp.log(jnp.sum(jnp.exp(z), axis=1, keepdims=True))


def _lane_iota():
    return lax.iota(jnp.int32, 16)


def _permute(v, idx):
    dn = lax.GatherDimensionNumbers(offset_dims=(), collapsed_slice_dims=(0,),
                                    start_index_map=(0,))
    return lax.gather(v, idx[:, None], dn, slice_sizes=(1,),
                      mode=lax.GatherScatterMode.PROMISE_IN_BOUNDS)


def _unpack2(hrows, e, off):
    u = hrows[e, pl.ds(off, 32)]
    return plsc.unpack(u, format=plsc.PackFormat.INTERLEAVED,
                       preferred_element_type=jnp.float32)


def _zero_acc(zbuf, acc_sh, width, row0):
    def zrow(i, carry):
        for k in range(width // 16):
            zbuf[i, pl.ds(16 * k, 16)] = jnp.zeros((16,), jnp.float32)
        return carry
    lax.fori_loop(0, ZR, zrow, 0)
    for j in range(RPT // ZR):
        pltpu.sync_copy(zbuf, acc_sh.at[pl.ds(row0 + j * ZR, ZR)])


def _writeback(acc_sh, out_hbm, cid, row0):
    for j in range(RPT // ZR):
        pltpu.sync_copy(acc_sh.at[pl.ds(row0 + j * ZR, ZR)],
                        out_hbm.at[cid, pl.ds(row0 + j * ZR, ZR)])


def _edge1(hrows, arows, drows, mbuf, e, consts):
    hi, hi8 = consts
    a = arows[e, pl.ds(0, 16)]
    vd = drows[e, pl.ds(0, 16)]
    vb = _permute(vd, hi8)
    ex = jnp.exp(_leaky(a + vd) - vb)
    u0a, u0b = _unpack2(hrows, e, 0)
    u1a, u1b = _unpack2(hrows, e, 32)
    for k, u in enumerate((u0a, u0b, u1a, u1b)):
        mbuf[e, pl.ds(16 * k, 16)] = u * _permute(ex, 2 * k + hi)
    mbuf[e, pl.ds(64, 16)] = ex


def _edge2(hrows, arows, drows, mbuf, e, consts):
    c0, c1 = consts
    va = arows[e, pl.ds(0, 16)]
    vd = drows[e, pl.ds(0, 16)]
    exb = jnp.exp(_leaky(_permute(va, c1) + _permute(vd, c0))
                  - _permute(vd, c1))
    u0a, u0b = _unpack2(hrows, e, 0)
    u1a, _ = _unpack2(hrows, e, 32)
    mbuf[e, pl.ds(0, 16)] = u0a * exb
    mbuf[e, pl.ds(16, 16)] = u0b * exb
    mbuf[e, pl.ds(32, 16)] = u1a * exb
    mbuf[e, pl.ds(48, 16)] = va * exb


def _consts1():
    lanes = _lane_iota()
    return jnp.where(lanes >= 8, 1, 0), (lanes % 8) + 8


def _consts2():
    lanes = _lane_iota()
    return lanes * 0, lanes * 0 + 1


def _make_sc_body(edge_fn, make_consts, acc_w):
    def body(tsh_hbm, tsa_hbm, td_hbm, src_hbm, dst_hbm, out_hbm,
             src_all, dst_all, hrows0, hrows1, hrows2, hrows3,
             arows0, arows1, arows2, arows3, drows0, drows1, drows2, drows3,
             mbuf0, mbuf1, mbuf2, mbuf3, zbuf, acc_sh,
             sh0, sh1, sh2, sh3, sa0, sa1, sa2, sa3, sd0, sd1, sd2, sd3,
             sm0, sm1, sm2, sm3):
        cid = lax.axis_index("c")
        sid = lax.axis_index("s")
        wid = sid * NC + cid
        row0 = sid * RPT
        _zero_acc(zbuf, acc_sh, acc_w, row0)
        pltpu.sync_copy(src_hbm.at[wid], src_all)
        pltpu.sync_copy(dst_hbm.at[wid], dst_all)
        plsc.subcore_barrier()

        consts = make_consts()
        hrows = [hrows0, hrows1, hrows2, hrows3]
        arows = [arows0, arows1, arows2, arows3]
        drows = [drows0, drows1, drows2, drows3]
        mbufs = [mbuf0, mbuf1, mbuf2, mbuf3]
        sems_h = [sh0, sh1, sh2, sh3]
        sems_a = [sa0, sa1, sa2, sa3]
        sems_d = [sd0, sd1, sd2, sd3]
        sems_m = [sm0, sm1, sm2, sm3]

        def fetch(c, b):
            pltpu.async_copy(tsh_hbm.at[src_all.at[c]], hrows[b], sems_h[b])
            pltpu.async_copy(tsa_hbm.at[src_all.at[c]], arows[b], sems_a[b])
            pltpu.async_copy(td_hbm.at[dst_all.at[c]], drows[b], sems_d[b])

        def wait(c, b):
            pltpu.make_async_copy(tsh_hbm.at[src_all.at[c]], hrows[b],
                                  sems_h[b]).wait()
            pltpu.make_async_copy(tsa_hbm.at[src_all.at[c]], arows[b],
                                  sems_a[b]).wait()
            pltpu.make_async_copy(td_hbm.at[dst_all.at[c]], drows[b],
                                  sems_d[b]).wait()

        def wait_scatter(c, b):
            pltpu.make_async_copy(mbufs[b], acc_sh.at[dst_all.at[c]],
                                  sems_m[b]).wait()

        for p in range(3):
            fetch(p, p)

        def loop(g, carry):
            for b in range(4):
                c = 4 * g + b

                @pl.when(c < NCHUNK)
                def _():
                    @pl.when(c + 3 < NCHUNK)
                    def _():
                        fetch(c + 3, (b + 3) % 4)
                    wait(c, b)

                    @pl.when(c >= 4)
                    def _():
                        wait_scatter(c, b)

                    @plsc.parallel_loop(0, CH, unroll=4)
                    def _(e):
                        edge_fn(hrows[b], arows[b], drows[b], mbufs[b], e,
                                consts)
                    pltpu.async_copy(mbufs[b], acc_sh.at[dst_all.at[c]],
                                     sems_m[b], add=True)
            return carry
        lax.fori_loop(0, (NCHUNK + 3) // 4, loop, 0)
        for p in range(4):
            c = NCHUNK - 1 - p
            wait_scatter(c, c % 4)
        plsc.subcore_barrier()
        _writeback(acc_sh, out_hbm, cid, row0)
    return body


_mesh = plsc.VectorSubcoreMesh(core_axis_name="c", subcore_axis_name="s")


def _make_sc(edge_fn, make_consts, acc_w):
    body = _make_sc_body(edge_fn, make_consts, acc_w)
    return functools.partial(
        pl.kernel,
        out_type=jax.ShapeDtypeStruct((NC, N, acc_w), jnp.float32),
        mesh=_mesh,
        scratch_types=[
            pltpu.VMEM((NCHUNK, CH), jnp.int32),
            pltpu.VMEM((NCHUNK, CH), jnp.int32),
        ] + [pltpu.VMEM((CH, 64), jnp.bfloat16)] * 4
          + [pltpu.VMEM((CH, 16), jnp.float32)] * 8
          + [pltpu.VMEM((CH, acc_w), jnp.float32)] * 4
          + [
            pltpu.VMEM((ZR, acc_w), jnp.float32),
            pltpu.VMEM_SHARED((N, acc_w), jnp.float32),
        ] + [pltpu.SemaphoreType.DMA] * 16,
        compiler_params=pltpu.CompilerParams(use_tc_tiling_on_sc=False,
                                             needs_layout_passes=False),
    )(body)


_sc1 = _make_sc(_edge1, _consts1, ACC_W1)
_sc2 = _make_sc(_edge2, _consts2, ACC_W2)


def kernel(x, edge_index, W1, att_src1, att_dst1, b1, W2, att_src2, att_dst2, b2):
    f32 = jnp.float32
    perm = jnp.array(_PERM, dtype=jnp.int32)
    eye_h = jnp.eye(H, dtype=f32)
    As = (att_src1[:, :, None] * eye_h[:, None, :]).reshape(H * C1, H)
    Ad = (att_dst1[:, :, None] * eye_h[:, None, :]).reshape(H * C1, H)
    W1p = W1[:, perm]
    Asp = As[perm, :]
    Adp = Ad[perm, :]
    R8 = jnp.kron(eye_h, jnp.ones((1, C1), dtype=f32))
    W2pad = jnp.pad(W2, ((0, 0), (0, 64 - NCLS)))
    W2p = W2pad[:, perm]
    as2 = jnp.pad(att_src2, ((0, 0), (0, 64 - NCLS)))[:, perm]
    ad2 = jnp.pad(att_dst2, ((0, 0), (0, 64 - NCLS)))[:, perm]
    pinv1 = jnp.asarray(_PINV1)
    pinv2 = jnp.asarray(_PINV2)
    b1r = b1.reshape(1, H * C1)
    b2r = b2.reshape(1, NCLS)
    src = edge_index[0].reshape(NW, NCHUNK, CH)
    dst = edge_index[1].reshape(NW, NCHUNK, CH)

    tsh1, tsa1, td1 = pl.pallas_call(
        _tc1_body,
        out_shape=[jax.ShapeDtypeStruct((N, 64), jnp.bfloat16),
                   jax.ShapeDtypeStruct((N, 16), f32),
                   jax.ShapeDtypeStruct((N, 16), f32)],
    )(x, W1p, Asp, Adp)

    acc1 = _sc1(tsh1, tsa1, td1, src, dst)

    tsh2, tsa2, td2 = pl.pallas_call(
        _tc2_body,
        out_shape=[jax.ShapeDtypeStruct((N, 64), jnp.bfloat16),
                   jax.ShapeDtypeStruct((N, 16), f32),
                   jax.ShapeDtypeStruct((N, 16), f32)],
    )(acc1, tsh1, tsa1, td1, b1r, W2p, as2, ad2, R8, pinv1)

    acc2 = _sc2(tsh2, tsa2, td2, src, dst)

    out = pl.pallas_call(
        _tc3_body,
        out_shape=jax.ShapeDtypeStruct((N, NCLS), f32),
    )(acc2, tsh2, tsa2, td2, b2r, pinv2)
    return out
